# baseline (speedup 1.0000x reference)
import jax
import jax.numpy as jnp
from jax import lax
from jax.experimental import pallas as pl
from jax.experimental.pallas import tpu as pltpu
from jax.experimental.pallas import tpu_sc as plsc

MAX_LEN = 8192
D_MODEL = 768
LANES = 16
NUM_CORES = 2
NUM_SUBCORES = 16
NUM_WORKERS = NUM_CORES * NUM_SUBCORES
ROWS_PER_WORKER = MAX_LEN // NUM_WORKERS
CHUNK = 16
NUM_CHUNKS = ROWS_PER_WORKER // CHUNK
NBUF = 3
VECS_PER_ROW = D_MODEL // LANES
COMB_ROWS = 24 * 8


def _body(hours_hbm, days_hbm, pe_hbm, htab_hbm, dtab_hbm, out_hbm,
          hidx_v, didx_v, cidx_v, hloc_v, dtab_v, bloc_v, comb_hbm,
          bufs, gbufs, sem_in, sem_out, sem_g, stg0, stg1, stg2, stg3):
    cid = lax.axis_index("c")
    sid = lax.axis_index("s")
    wid = sid * NUM_CORES + cid
    base = wid * ROWS_PER_WORKER

    h2 = lax.min(sid + 16, 23)
    stage = [
        pltpu.async_copy(hours_hbm.at[pl.ds(base, ROWS_PER_WORKER)],
                         hidx_v, stg0),
        pltpu.async_copy(days_hbm.at[pl.ds(base, ROWS_PER_WORKER)],
                         didx_v, stg1),
        pltpu.async_copy(dtab_hbm, dtab_v, stg2),
        pltpu.async_copy(
            htab_hbm.at[pl.ds(pl.multiple_of(sid * D_MODEL, 8), D_MODEL)],
            hloc_v.at[pl.ds(0, D_MODEL)], stg3),
        pltpu.async_copy(
            htab_hbm.at[pl.ds(pl.multiple_of(h2 * D_MODEL, 8), D_MODEL)],
            hloc_v.at[pl.ds(D_MODEL, D_MODEL)], stg3),
    ]

    def in_copy(c, slot, start):
        mk = pltpu.async_copy if start else pltpu.make_async_copy
        return mk(pe_hbm.at[pl.ds(base + c * CHUNK, CHUNK)], bufs.at[slot],
                  sem_in.at[slot])

    def out_copy(c, slot, start):
        mk = pltpu.async_copy if start else pltpu.make_async_copy
        return mk(bufs.at[slot], out_hbm.at[pl.ds(base + c * CHUNK, CHUNK)],
                  sem_out.at[slot])

    def g_copy(c, slot, start):
        mk = pltpu.async_copy if start else pltpu.make_async_copy
        idx = cidx_v.at[pl.ds(pl.multiple_of(c * CHUNK, CHUNK), CHUNK)]
        return mk(comb_hbm.at[idx], gbufs.at[slot], sem_g.at[slot])

    for k in range(NBUF - 1):
        in_copy(k, k, start=True)

    stage[2].wait()
    stage[3].wait()
    stage[3].wait()

    def build_block(blk, half):
        for i in range(8):
            di = min(i, 6)

            @plsc.parallel_loop(0, VECS_PER_ROW, 1, unroll=4)
            def build_body(j, _i=i, _di=di, _half=half):
                o = pl.multiple_of(j * LANES, LANES)
                ho = pl.multiple_of(_half * D_MODEL + o, LANES)
                s = pl.ds(o, LANES)
                bloc_v[_i, s] = hloc_v[pl.ds(ho, LANES)] + dtab_v[_di, s]
        pltpu.sync_copy(
            bloc_v, comb_hbm.at[pl.ds(pl.multiple_of(blk * 8, 8), 8)])

    build_block(sid, 0)

    @pl.when(sid < 8)
    def _second_block():
        build_block(sid + 16, 1)
    plsc.subcore_barrier()

    stage[0].wait()
    stage[1].wait()
    for g in range(ROWS_PER_WORKER // LANES):
        o = pl.multiple_of(g * LANES, LANES)
        s = pl.ds(o, LANES)
        cidx_v[s] = lax.shift_left(hidx_v[s], 3) + didx_v[s]

    for k in range(NBUF - 1):
        g_copy(k, k, start=True)

    def chunk_body(t, carry):
        slot = lax.rem(t, NBUF)
        in_copy(t, slot, start=False).wait()
        g_copy(t, slot, start=False).wait()
        for r in range(0, CHUNK, 4):

            @plsc.parallel_loop(0, VECS_PER_ROW, 1, unroll=1)
            def vec_body(j, _r=r):
                s = pl.ds(pl.multiple_of(j * LANES, LANES), LANES)
                for i in range(4):
                    bufs[slot, _r + i, s] = (bufs[slot, _r + i, s]
                                             + gbufs[slot, _r + i, s])
        out_copy(t, slot, start=True)
        nslot = lax.rem(slot + NBUF - 1, NBUF)

        @pl.when(t >= 1)
        def _wait_prev():
            out_copy(t - 1, nslot, start=False).wait()

        @pl.when(t + NBUF - 1 < NUM_CHUNKS)
        def _start_next():
            in_copy(t + NBUF - 1, nslot, start=True)
            g_copy(t + NBUF - 1, nslot, start=True)
        return carry

    lax.fori_loop(0, NUM_CHUNKS, chunk_body, 0)
    out_copy(NUM_CHUNKS - 1, (NUM_CHUNKS - 1) % NBUF, start=False).wait()


@jax.jit
def _temporal_embedding(hours, days, pe, hour_embedding, day_embedding):
    mesh = plsc.VectorSubcoreMesh(
        core_axis_name="c", subcore_axis_name="s",
        num_cores=NUM_CORES, num_subcores=NUM_SUBCORES)
    return pl.kernel(
        _body,
        out_type=jax.ShapeDtypeStruct((MAX_LEN, D_MODEL), jnp.float32),
        mesh=mesh,
        scratch_types=[
            pltpu.VMEM((ROWS_PER_WORKER,), jnp.int32),
            pltpu.VMEM((ROWS_PER_WORKER,), jnp.int32),
            pltpu.VMEM((ROWS_PER_WORKER,), jnp.int32),
            pltpu.VMEM((2 * D_MODEL,), jnp.float32),
            pltpu.VMEM((7, D_MODEL), jnp.float32),
            pltpu.VMEM((8, D_MODEL), jnp.float32),
            pltpu.HBM((COMB_ROWS, D_MODEL), jnp.float32),
            pltpu.VMEM((NBUF, CHUNK, D_MODEL), jnp.float32),
            pltpu.VMEM((NBUF, CHUNK, D_MODEL), jnp.float32),
            pltpu.SemaphoreType.DMA((NBUF,)),
            pltpu.SemaphoreType.DMA((NBUF,)),
            pltpu.SemaphoreType.DMA((NBUF,)),
            pltpu.SemaphoreType.DMA,
            pltpu.SemaphoreType.DMA,
            pltpu.SemaphoreType.DMA,
            pltpu.SemaphoreType.DMA,
        ],
    )(hours, days, pe, hour_embedding.reshape(-1), day_embedding)


def kernel(hours, days, pe, hour_embedding, day_embedding):
    return _temporal_embedding(hours, days, pe, hour_embedding, day_embedding)


# speedup vs baseline: 1.1926x; 1.1926x over previous
import jax
import jax.numpy as jnp
from jax import lax
from jax.experimental import pallas as pl
from jax.experimental.pallas import tpu as pltpu
from jax.experimental.pallas import tpu_sc as plsc

MAX_LEN = 8192
D_MODEL = 768
LANES = 16
NUM_CORES = 2
NUM_SUBCORES = 16
NUM_WORKERS = NUM_CORES * NUM_SUBCORES
ROWS_PER_WORKER = MAX_LEN // NUM_WORKERS
CHUNK = 16
NUM_CHUNKS = ROWS_PER_WORKER // CHUNK
NBUF = 4
VECS_PER_ROW = D_MODEL // LANES


def _body(hours_hbm, days_hbm, pe_hbm, htab_hbm, dtab_hbm, out_hbm,
          hidx_v, didx_v, htab_v, dtab_v, bufs, sem_in, sem_out,
          stg0, stg1, stg2, stg3):
    wid = lax.axis_index("s") * NUM_CORES + lax.axis_index("c")
    base = wid * ROWS_PER_WORKER

    stage = [
        pltpu.async_copy(hours_hbm.at[pl.ds(base, ROWS_PER_WORKER)],
                         hidx_v, stg0),
        pltpu.async_copy(days_hbm.at[pl.ds(base, ROWS_PER_WORKER)],
                         didx_v, stg1),
        pltpu.async_copy(htab_hbm, htab_v, stg2),
        pltpu.async_copy(dtab_hbm, dtab_v, stg3),
    ]

    def in_copy(c, slot, start):
        mk = pltpu.async_copy if start else pltpu.make_async_copy
        return mk(pe_hbm.at[pl.ds(base + c * CHUNK, CHUNK)], bufs.at[slot],
                  sem_in.at[slot])

    def out_copy(c, slot, start):
        mk = pltpu.async_copy if start else pltpu.make_async_copy
        return mk(bufs.at[slot], out_hbm.at[pl.ds(base + c * CHUNK, CHUNK)],
                  sem_out.at[slot])

    for k in range(NBUF - 1):
        in_copy(k, k, start=True)
    for cp in stage:
        cp.wait()

    def chunk_body(t, carry):
        slot = lax.rem(t, NBUF)
        in_copy(t, slot, start=False).wait()
        off = pl.multiple_of(t * CHUNK, CHUNK)
        hvec = hidx_v[pl.ds(off, LANES)]
        dvec = didx_v[pl.ds(off, LANES)]
        for r in range(0, CHUNK, 4):
            hh = [hvec[r + i] for i in range(4)]
            dd = [dvec[r + i] for i in range(4)]

            @plsc.parallel_loop(0, VECS_PER_ROW, 1, unroll=1)
            def vec_body(j, _r=r, _hh=hh, _dd=dd):
                s = pl.ds(pl.multiple_of(j * LANES, LANES), LANES)
                for i in range(4):
                    bufs[slot, _r + i, s] = (bufs[slot, _r + i, s]
                                             + htab_v[_hh[i], s]
                                             + dtab_v[_dd[i], s])
        out_copy(t, slot, start=True)
        nslot = lax.rem(slot + NBUF - 1, NBUF)

        @pl.when(t >= 1)
        def _wait_prev():
            out_copy(t - 1, nslot, start=False).wait()

        @pl.when(t + NBUF - 1 < NUM_CHUNKS)
        def _start_next():
            in_copy(t + NBUF - 1, nslot, start=True)
        return carry

    lax.fori_loop(0, NUM_CHUNKS, chunk_body, 0)
    out_copy(NUM_CHUNKS - 1, (NUM_CHUNKS - 1) % NBUF, start=False).wait()


@jax.jit
def _temporal_embedding(hours, days, pe, hour_embedding, day_embedding):
    mesh = plsc.VectorSubcoreMesh(
        core_axis_name="c", subcore_axis_name="s",
        num_cores=NUM_CORES, num_subcores=NUM_SUBCORES)
    return pl.kernel(
        _body,
        out_type=jax.ShapeDtypeStruct((MAX_LEN, D_MODEL), jnp.float32),
        mesh=mesh,
        scratch_types=[
            pltpu.VMEM((ROWS_PER_WORKER,), jnp.int32),
            pltpu.VMEM((ROWS_PER_WORKER,), jnp.int32),
            pltpu.VMEM((24, D_MODEL), jnp.float32),
            pltpu.VMEM((7, D_MODEL), jnp.float32),
            pltpu.VMEM((NBUF, CHUNK, D_MODEL), jnp.float32),
            pltpu.SemaphoreType.DMA((NBUF,)),
            pltpu.SemaphoreType.DMA((NBUF,)),
            pltpu.SemaphoreType.DMA,
            pltpu.SemaphoreType.DMA,
            pltpu.SemaphoreType.DMA,
            pltpu.SemaphoreType.DMA,
        ],
    )(hours, days, pe, hour_embedding, day_embedding)


def kernel(hours, days, pe, hour_embedding, day_embedding):
    return _temporal_embedding(hours, days, pe, hour_embedding, day_embedding)
